# baseline (speedup 1.0000x reference)
import functools

import jax
import jax.numpy as jnp
from jax import lax
from jax.experimental import pallas as pl
from jax.experimental.pallas import tpu as pltpu
from jax.experimental.pallas import tpu_sc as plsc

NUM_CLASSES = 100000
EMBED_DIM = 64
BATCH = 16384

_NC = 2
_NS = 16
_NW = _NC * _NS
_ROWS_PER_W = BATCH // _NW
_GCHUNK = 128
_NG = _ROWS_PER_W // _GCHUNK


def _center_loss_body(emb_hbm, tgt_hbm, tbl_hbm, out_hbm,
                      idx_v, emb_v, rows_v, out_v, gsem, esem):
    wid = lax.axis_index("s") * _NC + lax.axis_index("c")
    base = wid * _ROWS_PER_W

    pltpu.sync_copy(tgt_hbm.at[wid], idx_v)

    gathers = []
    for j in range(_NG):
        gathers.append(pltpu.async_copy(
            tbl_hbm.at[idx_v.at[j]],
            rows_v.at[pl.ds(j * _GCHUNK, _GCHUNK)],
            gsem))
    emb_cp = pltpu.async_copy(emb_hbm.at[pl.ds(base, _ROWS_PER_W)],
                              emb_v, esem)
    emb_cp.wait()
    for g in gathers:
        g.wait()

    zero = jnp.zeros((16,), jnp.float32)

    def body(r, accs):
        new = []
        for j in range(4):
            e = emb_v[r, pl.ds(j * 16, 16)]
            c = rows_v[r, pl.ds(j * 16, 16)]
            d = e - c
            new.append(accs[j] + d * d)
        return tuple(new)

    accs = lax.fori_loop(0, _ROWS_PER_W, body, (zero, zero, zero, zero))
    total = (accs[0] + accs[1]) + (accs[2] + accs[3])
    out_v[...] = total * jnp.float32(1.0 / BATCH)
    pltpu.sync_copy(out_v, out_hbm.at[wid])


@jax.jit
def _center_loss(embeddings, targets, centers):
    tgt = targets.astype(jnp.int32).reshape(_NW, _NG, _GCHUNK)
    mesh = plsc.VectorSubcoreMesh(core_axis_name="c", subcore_axis_name="s")
    partials = pl.kernel(
        _center_loss_body,
        mesh=mesh,
        out_type=jax.ShapeDtypeStruct((_NW, 16), jnp.float32),
        scratch_types=[
            pltpu.VMEM((_NG, _GCHUNK), jnp.int32),
            pltpu.VMEM((_ROWS_PER_W, EMBED_DIM), jnp.float32),
            pltpu.VMEM((_ROWS_PER_W, EMBED_DIM), jnp.float32),
            pltpu.VMEM((16,), jnp.float32),
            pltpu.SemaphoreType.DMA,
            pltpu.SemaphoreType.DMA,
        ],
        compiler_params=pltpu.CompilerParams(use_tc_tiling_on_sc=False),
    )(embeddings, tgt, centers)
    return jnp.sum(partials)


def kernel(embeddings, targets, centers):
    return _center_loss(embeddings, targets, centers)
